# baseline (speedup 1.0000x reference)
import functools

import jax
import jax.numpy as jnp
from jax import lax
from jax.experimental import pallas as pl
from jax.experimental.pallas import tpu as pltpu
from jax.experimental.pallas import tpu_sc as plsc

_NUM_CORES = 2
_NUM_SUBCORES = 16
_NW = _NUM_CORES * _NUM_SUBCORES
_BT = 128


def _make_gather(Vp, D, L, B):
    nbt = B // _BT
    assert nbt == _NW and D == 64 and L % 2 == 0
    bpw = L * _BT
    mesh = plsc.VectorSubcoreMesh(core_axis_name="c", subcore_axis_name="s")

    @functools.partial(
        pl.kernel,
        mesh=mesh,
        out_type=jax.ShapeDtypeStruct((L, 8, nbt, 8, _BT), jnp.float32),
        compiler_params=pltpu.CompilerParams(
            use_tc_tiling_on_sc=False, needs_layout_passes=False
        ),
        scratch_types=[
            pltpu.VMEM((bpw,), jnp.int32),
            pltpu.VMEM((_BT, D), jnp.float32),
            pltpu.VMEM((_BT, D), jnp.float32),
            pltpu.VMEM((8, 8, _BT), jnp.float32),
            pltpu.VMEM((8, 8, _BT), jnp.float32),
            pltpu.SemaphoreType.DMA,
            pltpu.SemaphoreType.DMA,
            pltpu.SemaphoreType.DMA,
            pltpu.SemaphoreType.DMA,
        ],
    )
    def k(table_hbm, idx_hbm, out_hbm, idx_v, rows0, rows1, tb0, tb1,
          g0, g1, s0, s1):
        w = lax.axis_index("s") * _NUM_CORES + lax.axis_index("c")
        pltpu.sync_copy(idx_hbm.at[pl.ds(w * bpw, bpw)], idx_v)

        iota = lax.iota(jnp.int32, 16)
        bvecs = [g * 16 + iota for g in range(8)]
        banks = ((rows0, tb0, g0, s0), (rows1, tb1, g1, s1))

        def transpose_block(rowsb, tbb):
            for d in range(D):
                dspl = jnp.full((16,), d, jnp.int32)
                for g in range(8):
                    v = plsc.load_gather(rowsb, [bvecs[g], dspl])
                    tbb[d // 8, d % 8, pl.ds(g * 16, 16)] = v

        def body(i, carry):
            descs = []
            for kb in range(2):
                rowsb, tbb, gs, ss = banks[kb]
                l = i * 2 + kb

                @pl.when(i > 0)
                def _():
                    pltpu.make_async_copy(
                        tbb, out_hbm.at[0, :, 0], ss
                    ).wait()

                descs.append(
                    pltpu.async_copy(
                        table_hbm.at[idx_v.at[pl.ds(l * _BT, _BT)]],
                        rowsb,
                        gs,
                    )
                )
            for kb in range(2):
                rowsb, tbb, gs, ss = banks[kb]
                l = i * 2 + kb
                descs[kb].wait()
                transpose_block(rowsb, tbb)
                pltpu.async_copy(tbb, out_hbm.at[l, :, w], ss)
            return carry

        lax.fori_loop(0, L // 2, body, 0)
        for rowsb, tbb, gs, ss in banks:
            pltpu.make_async_copy(tbb, out_hbm.at[0, :, 0], ss).wait()

    return k


def kernel(sequence, table):
    B, L = sequence.shape
    V, D = table.shape
    idx = (sequence.T.reshape(L, _NW, _BT).transpose(1, 0, 2) * 2).reshape(-1)
    idx = idx.astype(jnp.int32)
    table_p = jnp.pad(table, ((0, 0), (0, 128 - D))).reshape(2 * V, D)
    out = _make_gather(2 * V, D, L, B)(table_p, idx)
    return out.transpose(2, 4, 0, 1, 3).reshape(B, L, D)


# speedup vs baseline: 2.7919x; 2.7919x over previous
import functools

import jax
import jax.numpy as jnp
from jax import lax
from jax.experimental import pallas as pl
from jax.experimental.pallas import tpu as pltpu
from jax.experimental.pallas import tpu_sc as plsc

_NUM_CORES = 2
_NUM_SUBCORES = 16
_NW = _NUM_CORES * _NUM_SUBCORES
_BT = 128


def _make_gather(Vp, D, L, B):
    nbt = B // _BT
    assert nbt == _NW and D == 64 and L % 2 == 0
    bpw = L * _BT
    mesh = plsc.VectorSubcoreMesh(core_axis_name="c", subcore_axis_name="s")

    @functools.partial(
        pl.kernel,
        mesh=mesh,
        out_type=jax.ShapeDtypeStruct((L, 8, nbt, 8, _BT), jnp.float32),
        compiler_params=pltpu.CompilerParams(
            use_tc_tiling_on_sc=False, needs_layout_passes=False
        ),
        scratch_types=[
            pltpu.VMEM((bpw,), jnp.int32),
            pltpu.VMEM((_BT, D), jnp.float32),
            pltpu.VMEM((_BT, D), jnp.float32),
            pltpu.VMEM((8, 8, _BT), jnp.float32),
            pltpu.VMEM((8, 8, _BT), jnp.float32),
            pltpu.SemaphoreType.DMA,
            pltpu.SemaphoreType.DMA,
            pltpu.SemaphoreType.DMA,
            pltpu.SemaphoreType.DMA,
        ],
    )
    def k(table_hbm, idx_hbm, out_hbm, idx_v, rows0, rows1, tb0, tb1,
          g0, g1, s0, s1):
        w = lax.axis_index("s") * _NUM_CORES + lax.axis_index("c")
        pltpu.sync_copy(idx_hbm.at[pl.ds(w * bpw, bpw)], idx_v)

        iota = lax.iota(jnp.int32, 16)
        bvecs = [g * 16 + iota for g in range(8)]
        banks = ((rows0, tb0, g0, s0), (rows1, tb1, g1, s1))

        def transpose_block(rowsb, tbb):
            @functools.partial(plsc.parallel_loop, 0, D, unroll=8)
            def _(d):
                dspl = jnp.full((16,), 0, jnp.int32) + d
                for g in range(8):
                    v = plsc.load_gather(rowsb, [bvecs[g], dspl])
                    tbb[d // 8, d % 8, pl.ds(g * 16, 16)] = v

        def body(i, carry):
            descs = []
            for kb in range(2):
                rowsb, tbb, gs, ss = banks[kb]
                l = i * 2 + kb

                @pl.when(i > 0)
                def _():
                    pltpu.make_async_copy(
                        tbb, out_hbm.at[0, :, 0], ss
                    ).wait()

                descs.append(
                    pltpu.async_copy(
                        table_hbm.at[idx_v.at[pl.ds(l * _BT, _BT)]],
                        rowsb,
                        gs,
                    )
                )
            for kb in range(2):
                rowsb, tbb, gs, ss = banks[kb]
                l = i * 2 + kb
                descs[kb].wait()
                transpose_block(rowsb, tbb)
                pltpu.async_copy(tbb, out_hbm.at[l, :, w], ss)
            return carry

        lax.fori_loop(0, L // 2, body, 0)
        for rowsb, tbb, gs, ss in banks:
            pltpu.make_async_copy(tbb, out_hbm.at[0, :, 0], ss).wait()

    return k


def kernel(sequence, table):
    B, L = sequence.shape
    V, D = table.shape
    idx = (sequence.T.reshape(L, _NW, _BT).transpose(1, 0, 2) * 2).reshape(-1)
    idx = idx.astype(jnp.int32)
    table_p = jnp.pad(table, ((0, 0), (0, 128 - D))).reshape(2 * V, D)
    out = _make_gather(2 * V, D, L, B)(table_p, idx)
    return out.transpose(2, 4, 0, 1, 3).reshape(B, L, D)
